# baseline (speedup 1.0000x reference)
import functools

import jax
import jax.numpy as jnp
from jax import lax
from jax.experimental import pallas as pl
from jax.experimental.pallas import tpu as pltpu
from jax.experimental.pallas import tpu_sc as plsc

_L = 16
_G = 128
_C = 512
_P = 128


@functools.lru_cache(maxsize=None)
def _build_gather(n_rows: int, vocab2: int, dim: int, hist: int):
    info = plsc.get_sparse_core_info()
    nc, ns = info.num_cores, info.num_subcores
    nw = nc * ns
    assert n_rows % (2 * nw * _C) == 0 and dim % _L == 0
    rpw = n_rows // nw
    npair = rpw // (2 * _C)
    ng = _C // _G

    mesh = plsc.VectorSubcoreMesh(core_axis_name="c", subcore_axis_name="s")

    @functools.partial(
        pl.kernel,
        out_type=jax.ShapeDtypeStruct((n_rows, _P), jnp.float32),
        mesh=mesh,
        compiler_params=pltpu.CompilerParams(
            use_tc_tiling_on_sc=False, needs_layout_passes=False),
        scratch_types=[
            pltpu.VMEM((_C,), jnp.int32),
            pltpu.VMEM((_C, dim), jnp.float32),
            pltpu.VMEM((_C,), jnp.int32),
            pltpu.VMEM((_C, dim), jnp.float32),
            pltpu.VMEM((_L,), jnp.float32),
            pltpu.SemaphoreType.DMA,
            pltpu.SemaphoreType.DMA,
        ],
    )
    def body(idx_hbm, table_hbm, zs_hbm, out_hbm,
             idx_a, rows_a, idx_b, rows_b, zs_v, sem_a, sem_b):
        wid = lax.axis_index("s") * nc + lax.axis_index("c")
        w0 = wid * rpw
        pltpu.sync_copy(zs_hbm, zs_v)

        def issue(idx_v, rows_v, sem, base):
            pltpu.sync_copy(idx_hbm.at[pl.ds(base, _C)], idx_v)
            for k in range(ng):
                pltpu.async_copy(
                    table_hbm.at[idx_v.at[pl.ds(k * _G, _G)]],
                    rows_v.at[pl.ds(k * _G, _G)],
                    sem,
                )

        def finish(idx_v, rows_v, sem, base, first):
            for k in range(ng):
                pltpu.make_async_copy(
                    table_hbm.at[idx_v.at[pl.ds(k * _G, _G)]],
                    rows_v.at[pl.ds(k * _G, _G)],
                    sem,
                ).wait()

            @pl.when(first)
            def _fix():
                zs = zs_v[...]

                def rowfix(i, c2):
                    for k in range(dim // _L):
                        sl = pl.ds(k * _L, _L)
                        rows_v[i, sl] = rows_v[i, sl] * zs
                    return c2

                lax.fori_loop(0, hist, rowfix, 0)

            pltpu.sync_copy(rows_v,
                            out_hbm.at[pl.ds(base, _C), pl.ds(0, dim)])

        issue(idx_a, rows_a, sem_a, pl.multiple_of(w0, _C))

        def pair(j, carry):
            e_base = pl.multiple_of(w0 + (2 * j) * _C, _C)
            o_base = pl.multiple_of(w0 + (2 * j + 1) * _C, _C)
            issue(idx_b, rows_b, sem_b, o_base)
            finish(idx_a, rows_a, sem_a, e_base, (wid == 0) & (j == 0))

            @pl.when(j < npair - 1)
            def _next():
                issue(idx_a, rows_a, sem_a,
                      pl.multiple_of(w0 + (2 * j + 2) * _C, _C))

            finish(idx_b, rows_b, sem_b, o_base, False)
            return carry

        lax.fori_loop(0, npair, pair, 0)

    return body


def kernel(y, table, is_train):
    b, h = y.shape
    vocab, dim = table.shape
    flat = y.reshape(-1).astype(jnp.int32)
    shifted = jnp.concatenate([jnp.zeros((h,), jnp.int32), flat[:-h]])
    train = is_train != 0
    idx = jnp.where(train, shifted, flat) * 2
    zscale = jnp.where(train, jnp.zeros((_L,), jnp.float32),
                       jnp.ones((_L,), jnp.float32))
    table2 = jnp.pad(table, ((0, 0), (0, _P - dim))).reshape(2 * vocab, dim)
    out128 = _build_gather(b * h, 2 * vocab, dim, h)(idx, table2, zscale)
    return out128.reshape(b, h, _P)[:, :, :dim]
